# Initial kernel scaffold; baseline (speedup 1.0000x reference)
import jax
import jax.numpy as jnp
from jax.experimental import pallas as pl


def kernel(input_ids, weight):
    raise NotImplementedError("write your pallas kernel here")


# speedup vs baseline: 1.3193x; 1.3193x over previous
import functools
import math

import jax
import jax.numpy as jnp
from jax import lax
from jax.experimental import pallas as pl
from jax.experimental.pallas import tpu as pltpu
from jax.experimental.pallas import tpu_sc as plsc

_VOCAB = 100000
_D = 1024
_L = 16
_NC = 2
_NS = 16
_NW = _NC * _NS
_C = 32
_SCALE = math.sqrt(_D)


@functools.partial(jax.jit, static_argnames=("n_rows",))
def _gather_scale(idx, weight, n_rows):
    n_chunks = n_rows // (_NW * _C)
    mesh = plsc.VectorSubcoreMesh(core_axis_name="c", subcore_axis_name="s")

    @functools.partial(
        pl.kernel,
        out_type=jax.ShapeDtypeStruct((n_rows, _D), jnp.float32),
        mesh=mesh,
        scratch_types=[
            pltpu.VMEM((n_chunks, _C), jnp.int32),
            pltpu.VMEM((_C, _D), jnp.float32),
            pltpu.VMEM((_C, _D), jnp.float32),
            pltpu.SemaphoreType.DMA,
            pltpu.SemaphoreType.DMA,
            pltpu.SemaphoreType.DMA,
            pltpu.SemaphoreType.DMA,
        ],
    )
    def body(idx_hbm, w_hbm, out_hbm, idx_v, buf0, buf1, g0, g1, s0, s1):
        wid = lax.axis_index("s") * _NC + lax.axis_index("c")
        base = wid * (n_chunks * _C)
        pltpu.sync_copy(idx_hbm.at[wid], idx_v)

        bufs = (buf0, buf1)
        gsems = (g0, g1)
        ssems = (s0, s1)

        def gather(j):
            slot = j % 2
            return pltpu.async_copy(w_hbm.at[idx_v.at[j]], bufs[slot], gsems[slot])

        def scatter(j):
            slot = j % 2
            return pltpu.async_copy(
                bufs[slot], out_hbm.at[pl.ds(base + j * _C, _C)], ssems[slot]
            )

        def scale(j):
            buf = bufs[j % 2]

            def row(r, carry):
                for i in range(_D // _L):
                    sl = pl.ds(i * _L, _L)
                    buf[r, sl] = buf[r, sl] * _SCALE
                return carry

            lax.fori_loop(0, _C, row, 0)

        gathers = [None] * n_chunks
        scatters = [None] * n_chunks
        gathers[0] = gather(0)
        for j in range(n_chunks):
            if j + 1 < n_chunks:
                if j >= 1:
                    scatters[j - 1].wait()
                gathers[j + 1] = gather(j + 1)
            gathers[j].wait()
            scale(j)
            scatters[j] = scatter(j)
        if n_chunks >= 2:
            scatters[n_chunks - 2].wait()
        scatters[n_chunks - 1].wait()

    return body(idx, weight)


def kernel(input_ids, weight):
    b, s = input_ids.shape
    n_rows = b * s
    idx = input_ids.astype(jnp.int32).reshape(_NW, n_rows // (_NW * _C), _C)
    out = _gather_scale(idx, weight, n_rows)
    return out.reshape(b, s, _D)


# speedup vs baseline: 1.4792x; 1.1212x over previous
import functools
import math

import jax
import jax.numpy as jnp
from jax import lax
from jax.experimental import pallas as pl
from jax.experimental.pallas import tpu as pltpu
from jax.experimental.pallas import tpu_sc as plsc

_VOCAB = 100000
_D = 1024
_L = 16
_NC = 2
_NS = 16
_NW = _NC * _NS
_C = 32
_NBUF = 3
_SCALE = math.sqrt(_D)


@functools.partial(jax.jit, static_argnames=("n_rows",))
def _gather_scale(idx, weight, n_rows):
    n_chunks = n_rows // (_NW * _C)
    mesh = plsc.VectorSubcoreMesh(core_axis_name="c", subcore_axis_name="s")

    @functools.partial(
        pl.kernel,
        out_type=jax.ShapeDtypeStruct((n_rows, _D), jnp.float32),
        mesh=mesh,
        scratch_types=(
            [pltpu.VMEM((n_chunks, _C), jnp.int32)]
            + [pltpu.VMEM((_C, _D), jnp.float32)] * _NBUF
            + [pltpu.SemaphoreType.DMA] * (2 * _NBUF)
        ),
    )
    def body(idx_hbm, w_hbm, out_hbm, idx_v, *bufs_sems):
        bufs = bufs_sems[:_NBUF]
        gsems = bufs_sems[_NBUF : 2 * _NBUF]
        ssems = bufs_sems[2 * _NBUF :]
        wid = lax.axis_index("s") * _NC + lax.axis_index("c")
        base = wid * (n_chunks * _C)
        pltpu.sync_copy(idx_hbm.at[wid], idx_v)

        def gather(j):
            slot = j % _NBUF
            return pltpu.async_copy(w_hbm.at[idx_v.at[j]], bufs[slot], gsems[slot])

        def scatter(j):
            slot = j % _NBUF
            return pltpu.async_copy(
                bufs[slot], out_hbm.at[pl.ds(base + j * _C, _C)], ssems[slot]
            )

        def scale(j):
            buf = bufs[j % _NBUF]

            def row(r, carry):
                for i in range(_D // _L):
                    sl = pl.ds(i * _L, _L)
                    buf[r, sl] = buf[r, sl] * _SCALE
                return carry

            lax.fori_loop(0, _C, row, 0)

        gathers = [None] * n_chunks
        scatters = [None] * n_chunks
        for j in range(min(_NBUF - 1, n_chunks)):
            gathers[j] = gather(j)
        for j in range(n_chunks):
            gathers[j].wait()
            scale(j)
            scatters[j] = scatter(j)
            nxt = j + _NBUF - 1
            if nxt < n_chunks:
                if nxt - _NBUF >= 0:
                    scatters[nxt - _NBUF].wait()
                gathers[nxt] = gather(nxt)
        for j in range(max(0, n_chunks - _NBUF), n_chunks):
            if scatters[j] is not None:
                scatters[j].wait()

    return body(idx, weight)


def kernel(input_ids, weight):
    b, s = input_ids.shape
    n_rows = b * s
    idx = input_ids.astype(jnp.int32).reshape(_NW, n_rows // (_NW * _C), _C)
    out = _gather_scale(idx, weight, n_rows)
    return out.reshape(b, s, _D)
